# Initial kernel scaffold; baseline (speedup 1.0000x reference)
import jax
import jax.numpy as jnp
from jax.experimental import pallas as pl


def kernel(x, edge_index, W1, b1, W2, b2, Wg1, bg1, Wg2, bg2):
    raise NotImplementedError("write your pallas kernel here")


# speedup vs baseline: 15.3521x; 15.3521x over previous
import functools

import jax
import jax.numpy as jnp
from jax import lax
from jax.experimental import pallas as pl
from jax.experimental.pallas import tpu as pltpu
from jax.experimental.pallas import tpu_sc as plsc

_N = 10000
_E = 320000
_D = 128
_H = 64
_C = 2

_NC = 2
_NS = 16
_NW = _NC * _NS
_EPT = _E // _NW
_CH = 80
_NCHUNK = _EPT // _CH
_NPAD = 10240
_ROWS = _NPAD // _NS


def _make_edge_scatter(W, gather):
  mesh = plsc.VectorSubcoreMesh(core_axis_name="c", subcore_axis_name="s")

  @functools.partial(
      pl.kernel,
      out_type=jax.ShapeDtypeStruct((_NC, _NPAD, W), jnp.float32),
      mesh=mesh,
      compiler_params=pltpu.CompilerParams(use_tc_tiling_on_sc=False),
      scratch_types=[
          pltpu.VMEM((_CH,), jnp.int32),
          pltpu.VMEM((_CH,), jnp.int32),
          pltpu.VMEM((_CH, W), jnp.float32),
          pltpu.VMEM_SHARED((_NPAD, W), jnp.float32),
          pltpu.SemaphoreType.DMA,
      ],
  )
  def k(src_hbm, dst_hbm, rows_hbm, zeros_hbm, out_hbm,
        sidx, didx, rows_v, acc, sem):
    c = lax.axis_index("c")
    s = lax.axis_index("s")
    r0 = s * _ROWS
    pltpu.sync_copy(zeros_hbm.at[pl.ds(r0, _ROWS)], acc.at[pl.ds(r0, _ROWS)])
    if not gather:
      pltpu.sync_copy(rows_hbm, rows_v)
    plsc.subcore_barrier()

    base = (c * _NS + s) * _EPT

    def body(j, carry):
      off = base + j * _CH
      pltpu.sync_copy(dst_hbm.at[pl.ds(off, _CH)], didx)
      if gather:
        pltpu.sync_copy(src_hbm.at[pl.ds(off, _CH)], sidx)
        pltpu.async_copy(rows_hbm.at[sidx], rows_v, sem).wait()
      pltpu.sync_copy(rows_v, acc.at[didx], add=True)
      return carry

    lax.fori_loop(0, _NCHUNK, body, 0)
    plsc.subcore_barrier()
    pltpu.sync_copy(acc.at[pl.ds(r0, _ROWS)], out_hbm.at[c, pl.ds(r0, _ROWS)])

  return k


_deg_scatter = _make_edge_scatter(16, gather=False)
_pass1_scatter = _make_edge_scatter(_H, gather=True)
_pass2_scatter = _make_edge_scatter(16, gather=True)


def _tc_prep_body(x_ref, w1, b1r, w2, b2r, wg1, degp_ref,
                  y_ref, dinv_ref, mlp_ref):
  xv = x_ref[...]
  degs = degp_ref[0, :_N] + degp_ref[1, :_N]
  deg1 = degs[:, 0:1] + 1.0
  dinv = lax.rsqrt(jnp.maximum(deg1, 1e-12))
  xw = jnp.dot(xv, wg1[...], preferred_element_type=jnp.float32)
  y_ref[...] = xw * dinv
  dinv_ref[...] = dinv
  mh = jnp.maximum(
      jnp.dot(xv, w1[...], preferred_element_type=jnp.float32)
      + b1r[...][None, :], 0.0)
  mlp_ref[...] = (jnp.dot(mh, w2[...], preferred_element_type=jnp.float32)
                  + b2r[...][None, :])


_tc_prep = pl.pallas_call(
    _tc_prep_body,
    out_shape=[
        jax.ShapeDtypeStruct((_N, _H), jnp.float32),
        jax.ShapeDtypeStruct((_N, 1), jnp.float32),
        jax.ShapeDtypeStruct((_N, _C), jnp.float32),
    ],
)


def _tc_mid_body(hp_ref, y_ref, dinv_ref, bg1r, wg2, up_ref):
  hpre = hp_ref[0, :_N] + hp_ref[1, :_N] + y_ref[...]
  h = hpre * dinv_ref[...] + bg1r[...][None, :]
  hr = jnp.maximum(h, 0.0)
  u = jnp.dot(hr, wg2[...], preferred_element_type=jnp.float32)
  us = u * dinv_ref[...]
  up_ref[...] = jnp.concatenate(
      [us, jnp.zeros((_N, 16 - _C), jnp.float32)], axis=1)


_tc_mid = pl.pallas_call(
    _tc_mid_body,
    out_shape=jax.ShapeDtypeStruct((_N, 16), jnp.float32),
)


def _tc_final_body(sp_ref, up_ref, dinv_ref, mlp_ref, bg2r, out_ref):
  t = (sp_ref[0, :_N] + sp_ref[1, :_N] + up_ref[...]) * dinv_ref[...]
  o = mlp_ref[...] + t[:, 0:_C] + bg2r[...][None, :]
  m = jnp.max(o, axis=1, keepdims=True)
  lse = m + jnp.log(jnp.sum(jnp.exp(o - m), axis=1, keepdims=True))
  out_ref[...] = o - lse


_tc_final = pl.pallas_call(
    _tc_final_body,
    out_shape=jax.ShapeDtypeStruct((_N, _C), jnp.float32),
)


def kernel(x, edge_index, W1, b1, W2, b2, Wg1, bg1, Wg2, bg2):
  src = edge_index[0]
  dst = edge_index[1]
  zeros16 = jnp.zeros((_NPAD, 16), jnp.float32)
  zeros64 = jnp.zeros((_NPAD, _H), jnp.float32)
  ones = jnp.ones((_CH, 16), jnp.float32)

  degp = _deg_scatter(src, dst, ones, zeros16)
  y, dinv, mlp = _tc_prep(x, W1, b1, W2, b2, Wg1, degp)
  hp = _pass1_scatter(src, dst, y, zeros64)
  up = _tc_mid(hp, y, dinv, bg1, Wg2)
  sp = _pass2_scatter(src, dst, up, zeros16)
  return _tc_final(sp, up, dinv, mlp, bg2)


# speedup vs baseline: 44.2795x; 2.8843x over previous
import functools

import jax
import jax.numpy as jnp
from jax import lax
from jax.experimental import pallas as pl
from jax.experimental.pallas import tpu as pltpu
from jax.experimental.pallas import tpu_sc as plsc

_N = 10000
_E = 320000
_D = 128
_H = 64
_C = 2

_NC = 2
_NS = 16
_NW = _NC * _NS
_EPT = _E // _NW
_CH = 80
_NCHUNK = _EPT // _CH
_NPAD = 10240
_ROWS = _NPAD // _NS


_NB = 5
_NBLK = _NCHUNK // _NB
_NPAIR = (_NBLK - 1) // 2


def _make_edge_scatter(W, gather):
  mesh = plsc.VectorSubcoreMesh(core_axis_name="c", subcore_axis_name="s")

  @functools.partial(
      pl.kernel,
      out_type=jax.ShapeDtypeStruct((_NC, _NPAD, W), jnp.float32),
      mesh=mesh,
      compiler_params=pltpu.CompilerParams(use_tc_tiling_on_sc=False),
      scratch_types=[
          pltpu.VMEM((_NCHUNK, _CH), jnp.int32),
          pltpu.VMEM((_NCHUNK, _CH), jnp.int32),
          pltpu.VMEM((2, _NB, _CH, W), jnp.float32),
          pltpu.VMEM_SHARED((_NPAD, W), jnp.float32),
          pltpu.SemaphoreType.DMA,
          pltpu.SemaphoreType.DMA,
          pltpu.SemaphoreType.DMA,
      ],
  )
  def k(src2_hbm, dst2_hbm, rows_hbm, zeros_hbm, out_hbm,
        sidx2, didx2, rows, acc, gsem, ssem0, ssem1):
    c = lax.axis_index("c")
    s = lax.axis_index("s")
    r0 = s * _ROWS
    pltpu.sync_copy(zeros_hbm.at[pl.ds(r0, _ROWS)], acc.at[pl.ds(r0, _ROWS)])
    cbase = (c * _NS + s) * _NCHUNK
    if gather:
      pltpu.sync_copy(src2_hbm.at[pl.ds(cbase, _NCHUNK)], sidx2)
    else:
      pltpu.sync_copy(rows_hbm, rows.at[0, 0])
    pltpu.sync_copy(dst2_hbm.at[pl.ds(cbase, _NCHUNK)], didx2)
    plsc.subcore_barrier()

    def gather_block(setp, q0):
      hs = [pltpu.async_copy(rows_hbm.at[sidx2.at[q0 + b]],
                             rows.at[setp, b], gsem)
            for b in range(_NB)]
      for h in hs:
        h.wait()

    def fire_scatters(setp, q0, sem):
      for b in range(_NB):
        pltpu.async_copy(rows.at[setp, b], acc.at[didx2.at[q0 + b]],
                         sem, add=True)

    def drain_scatters(setp, sem):
      for b in range(_NB):
        pltpu.make_async_copy(zeros_hbm.at[pl.ds(0, _CH)],
                              rows.at[setp, b], sem).wait()

    if gather:
      def pair(i, carry):
        qa = (2 * i) * _NB
        qb = (2 * i + 1) * _NB

        @pl.when(i > 0)
        def _():
          drain_scatters(0, ssem0)
        gather_block(0, qa)
        fire_scatters(0, qa, ssem0)

        @pl.when(i > 0)
        def _():
          drain_scatters(1, ssem1)
        gather_block(1, qb)
        fire_scatters(1, qb, ssem1)
        return carry

      lax.fori_loop(0, _NPAIR, pair, 0)
      qe = (_NBLK - 1) * _NB
      drain_scatters(0, ssem0)
      gather_block(0, qe)
      fire_scatters(0, qe, ssem0)
      drain_scatters(0, ssem0)
      drain_scatters(1, ssem1)
    else:
      def blockd(i, carry):
        for b in range(_NB):
          pltpu.async_copy(rows.at[0, 0], acc.at[didx2.at[i * _NB + b]],
                           ssem0, add=True)

        @pl.when(i > 0)
        def _():
          for b in range(_NB):
            pltpu.make_async_copy(zeros_hbm.at[pl.ds(0, _CH)],
                                  rows.at[0, 0], ssem0).wait()
        return carry

      lax.fori_loop(0, _NBLK, blockd, 0)
      for b in range(_NB):
        pltpu.make_async_copy(zeros_hbm.at[pl.ds(0, _CH)],
                              rows.at[0, 0], ssem0).wait()

    plsc.subcore_barrier()
    pltpu.sync_copy(acc.at[pl.ds(r0, _ROWS)], out_hbm.at[c, pl.ds(r0, _ROWS)])

  return k


_deg_scatter = _make_edge_scatter(16, gather=False)
_pass1_scatter = _make_edge_scatter(_H, gather=True)
_pass2_scatter = _make_edge_scatter(16, gather=True)


def _tc_prep_body(x_ref, w1, b1r, w2, b2r, wg1, degp_ref,
                  y_ref, dinv_ref, mlp_ref):
  xv = x_ref[...]
  degs = degp_ref[0, :_N] + degp_ref[1, :_N]
  deg1 = degs[:, 0:1] + 1.0
  dinv = lax.rsqrt(jnp.maximum(deg1, 1e-12))
  xw = jnp.dot(xv, wg1[...], preferred_element_type=jnp.float32)
  y_ref[...] = xw * dinv
  dinv_ref[...] = dinv
  mh = jnp.maximum(
      jnp.dot(xv, w1[...], preferred_element_type=jnp.float32)
      + b1r[...][None, :], 0.0)
  mlp_ref[...] = (jnp.dot(mh, w2[...], preferred_element_type=jnp.float32)
                  + b2r[...][None, :])


_tc_prep = pl.pallas_call(
    _tc_prep_body,
    out_shape=[
        jax.ShapeDtypeStruct((_N, _H), jnp.float32),
        jax.ShapeDtypeStruct((_N, 1), jnp.float32),
        jax.ShapeDtypeStruct((_N, _C), jnp.float32),
    ],
)


def _tc_mid_body(hp_ref, y_ref, dinv_ref, bg1r, wg2, up_ref):
  hpre = hp_ref[0, :_N] + hp_ref[1, :_N] + y_ref[...]
  h = hpre * dinv_ref[...] + bg1r[...][None, :]
  hr = jnp.maximum(h, 0.0)
  u = jnp.dot(hr, wg2[...], preferred_element_type=jnp.float32)
  us = u * dinv_ref[...]
  up_ref[...] = jnp.concatenate(
      [us, jnp.zeros((_N, 16 - _C), jnp.float32)], axis=1)


_tc_mid = pl.pallas_call(
    _tc_mid_body,
    out_shape=jax.ShapeDtypeStruct((_N, 16), jnp.float32),
)


def _tc_final_body(sp_ref, up_ref, dinv_ref, mlp_ref, bg2r, out_ref):
  t = (sp_ref[0, :_N] + sp_ref[1, :_N] + up_ref[...]) * dinv_ref[...]
  o = mlp_ref[...] + t[:, 0:_C] + bg2r[...][None, :]
  m = jnp.max(o, axis=1, keepdims=True)
  lse = m + jnp.log(jnp.sum(jnp.exp(o - m), axis=1, keepdims=True))
  out_ref[...] = o - lse


_tc_final = pl.pallas_call(
    _tc_final_body,
    out_shape=jax.ShapeDtypeStruct((_N, _C), jnp.float32),
)


def kernel(x, edge_index, W1, b1, W2, b2, Wg1, bg1, Wg2, bg2):
  src2 = edge_index[0].reshape(_E // _CH, _CH)
  dst2 = edge_index[1].reshape(_E // _CH, _CH)
  zeros16 = jnp.zeros((_NPAD, 16), jnp.float32)
  zeros64 = jnp.zeros((_NPAD, _H), jnp.float32)
  ones = jnp.ones((_CH, 16), jnp.float32)

  degp = _deg_scatter(src2, dst2, ones, zeros16)
  y, dinv, mlp = _tc_prep(x, W1, b1, W2, b2, Wg1, degp)
  hp = _pass1_scatter(src2, dst2, y, zeros64)
  up = _tc_mid(hp, y, dinv, bg1, Wg2)
  sp = _pass2_scatter(src2, dst2, up, zeros16)
  return _tc_final(sp, up, dinv, mlp, bg2)


# speedup vs baseline: 46.4411x; 1.0488x over previous
import functools

import jax
import jax.numpy as jnp
from jax import lax
from jax.experimental import pallas as pl
from jax.experimental.pallas import tpu as pltpu
from jax.experimental.pallas import tpu_sc as plsc

_N = 10000
_E = 320000
_D = 128
_H = 64
_C = 2

_NC = 2
_NS = 16
_NW = _NC * _NS
_EPT = _E // _NW
_CH = 80
_NCHUNK = _EPT // _CH
_NPAD = 10240
_ROWS = _NPAD // _NS

_NB = 5
_NBLK = _NCHUNK // _NB
_NPAIR = (_NBLK - 1) // 2


def _make_edge_scatter(W, gather):
  mesh = plsc.VectorSubcoreMesh(core_axis_name="c", subcore_axis_name="s")
  lanes = W // 16

  @functools.partial(
      pl.kernel,
      out_type=jax.ShapeDtypeStruct((_NC, _NPAD, W), jnp.float32),
      mesh=mesh,
      compiler_params=pltpu.CompilerParams(use_tc_tiling_on_sc=False),
      scratch_types=[
          pltpu.VMEM((_EPT,), jnp.int32),
          pltpu.VMEM((_NCHUNK, _CH), jnp.int32),
          pltpu.VMEM((2, _NB, _CH, W), jnp.float32),
          pltpu.VMEM_SHARED((_NPAD, W), jnp.float32),
          pltpu.SemaphoreType.DMA,
          pltpu.SemaphoreType.DMA,
          pltpu.SemaphoreType.DMA,
      ],
  )
  def k(src_hbm, dst2_hbm, rows_hbm, out_hbm,
        sidx, didx2, rows, acc, gsem, ssem0, ssem1):
    c = lax.axis_index("c")
    s = lax.axis_index("s")
    tile = c * _NS + s
    r0 = s * _ROWS

    def zfill(j, carry):
      rows[1, 0, j // lanes, pl.ds((j % lanes) * 16, 16)] = (
          jnp.zeros((16,), jnp.float32))
      return carry

    lax.fori_loop(0, _CH * lanes, zfill, 0)
    zh = [pltpu.async_copy(rows.at[1, 0], acc.at[pl.ds(r0 + i * _CH, _CH)],
                           gsem)
          for i in range(_ROWS // _CH)]
    if gather:
      pltpu.sync_copy(src_hbm.at[pl.ds(tile * _EPT, _EPT)], sidx)
    else:
      pltpu.sync_copy(rows_hbm, rows.at[0, 0])
    pltpu.sync_copy(dst2_hbm.at[pl.ds(tile * _NCHUNK, _NCHUNK)], didx2)
    for h in zh:
      h.wait()
    plsc.subcore_barrier()

    def gather_block(setp, q0):
      hs = [pltpu.async_copy(rows_hbm.at[sidx.at[pl.ds((q0 + b) * _CH, _CH)]],
                             rows.at[setp, b], gsem)
            for b in range(_NB)]
      for h in hs:
        h.wait()

    def fire_scatters(setp, q0, sem):
      for b in range(_NB):
        pltpu.async_copy(rows.at[setp, b], acc.at[didx2.at[q0 + b]],
                         sem, add=True)

    def drain_scatters(setp, sem):
      dummy = rows_hbm.at[pl.ds(0, _CH)] if gather else rows_hbm
      for b in range(_NB):
        pltpu.make_async_copy(dummy, rows.at[setp, b], sem).wait()

    if gather:
      def pair(i, carry):
        qa = (2 * i) * _NB
        qb = (2 * i + 1) * _NB

        @pl.when(i > 0)
        def _():
          drain_scatters(0, ssem0)
        gather_block(0, qa)
        fire_scatters(0, qa, ssem0)

        @pl.when(i > 0)
        def _():
          drain_scatters(1, ssem1)
        gather_block(1, qb)
        fire_scatters(1, qb, ssem1)
        return carry

      lax.fori_loop(0, _NPAIR, pair, 0)
      qe = (_NBLK - 1) * _NB
      drain_scatters(0, ssem0)
      gather_block(0, qe)
      fire_scatters(0, qe, ssem0)
      drain_scatters(0, ssem0)
      drain_scatters(1, ssem1)
    else:
      def blockd(i, carry):
        for b in range(_NB):
          pltpu.async_copy(rows.at[0, 0], acc.at[didx2.at[i * _NB + b]],
                           ssem0, add=True)

        @pl.when(i > 0)
        def _():
          drain_scatters(0, ssem0)
        return carry

      lax.fori_loop(0, _NBLK, blockd, 0)
      drain_scatters(0, ssem0)

    plsc.subcore_barrier()
    pltpu.sync_copy(acc.at[pl.ds(r0, _ROWS)], out_hbm.at[c, pl.ds(r0, _ROWS)])

  return k


_deg_scatter = _make_edge_scatter(16, gather=False)
_pass1_scatter = _make_edge_scatter(_H, gather=True)
_pass2_scatter = _make_edge_scatter(16, gather=True)

_BN = 2000
_G = _N // _BN


def _tc_mm_body(x_ref, w1, b1r, w2, b2r, wg1, xw_ref, mlp_ref):
  xv = x_ref[...]
  xw_ref[...] = jnp.dot(xv, wg1[...], preferred_element_type=jnp.float32)
  mh = jnp.maximum(
      jnp.dot(xv, w1[...], preferred_element_type=jnp.float32) + b1r[...], 0.0)
  mlp_ref[...] = jnp.dot(mh, w2[...], preferred_element_type=jnp.float32) + b2r[...]


_tc_mm = pl.pallas_call(
    _tc_mm_body,
    grid=(_G,),
    in_specs=[
        pl.BlockSpec((_BN, _D), lambda i: (i, 0)),
        pl.BlockSpec((_D, _H), lambda i: (0, 0)),
        pl.BlockSpec((1, _H), lambda i: (0, 0)),
        pl.BlockSpec((_H, _C), lambda i: (0, 0)),
        pl.BlockSpec((1, _C), lambda i: (0, 0)),
        pl.BlockSpec((_D, _H), lambda i: (0, 0)),
    ],
    out_specs=[
        pl.BlockSpec((_BN, _H), lambda i: (i, 0)),
        pl.BlockSpec((_BN, _C), lambda i: (i, 0)),
    ],
    out_shape=[
        jax.ShapeDtypeStruct((_N, _H), jnp.float32),
        jax.ShapeDtypeStruct((_N, _C), jnp.float32),
    ],
)


def _tc_scale_body(degp_ref, xw_ref, y_ref, dinv_ref):
  degs = degp_ref[0] + degp_ref[1]
  deg1 = degs[:, 0:1] + 1.0
  dinv = lax.rsqrt(jnp.maximum(deg1, 1e-12))
  y_ref[...] = xw_ref[...] * dinv
  dinv_ref[...] = dinv


_tc_scale = pl.pallas_call(
    _tc_scale_body,
    grid=(_G,),
    in_specs=[
        pl.BlockSpec((2, _BN, 16), lambda i: (0, i, 0)),
        pl.BlockSpec((_BN, _H), lambda i: (i, 0)),
    ],
    out_specs=[
        pl.BlockSpec((_BN, _H), lambda i: (i, 0)),
        pl.BlockSpec((_BN, 1), lambda i: (i, 0)),
    ],
    out_shape=[
        jax.ShapeDtypeStruct((_N, _H), jnp.float32),
        jax.ShapeDtypeStruct((_N, 1), jnp.float32),
    ],
)


def _tc_mid_body(hp_ref, y_ref, dinv_ref, bg1r, wg2, up_ref):
  hpre = hp_ref[0] + hp_ref[1] + y_ref[...]
  h = hpre * dinv_ref[...] + bg1r[...]
  hr = jnp.maximum(h, 0.0)
  u = jnp.dot(hr, wg2[...], preferred_element_type=jnp.float32)
  us = u * dinv_ref[...]
  up_ref[...] = jnp.concatenate(
      [us, jnp.zeros((_BN, 16 - _C), jnp.float32)], axis=1)


_tc_mid = pl.pallas_call(
    _tc_mid_body,
    grid=(_G,),
    in_specs=[
        pl.BlockSpec((2, _BN, _H), lambda i: (0, i, 0)),
        pl.BlockSpec((_BN, _H), lambda i: (i, 0)),
        pl.BlockSpec((_BN, 1), lambda i: (i, 0)),
        pl.BlockSpec((1, _H), lambda i: (0, 0)),
        pl.BlockSpec((_H, _C), lambda i: (0, 0)),
    ],
    out_specs=pl.BlockSpec((_BN, 16), lambda i: (i, 0)),
    out_shape=jax.ShapeDtypeStruct((_N, 16), jnp.float32),
)


def _tc_final_body(sp_ref, up_ref, dinv_ref, mlp_ref, bg2r, out_ref):
  t = (sp_ref[0] + sp_ref[1] + up_ref[...]) * dinv_ref[...]
  o = mlp_ref[...] + t[:, 0:_C] + bg2r[...]
  m = jnp.max(o, axis=1, keepdims=True)
  lse = m + jnp.log(jnp.sum(jnp.exp(o - m), axis=1, keepdims=True))
  out_ref[...] = o - lse


_tc_final = pl.pallas_call(
    _tc_final_body,
    grid=(_G,),
    in_specs=[
        pl.BlockSpec((2, _BN, 16), lambda i: (0, i, 0)),
        pl.BlockSpec((_BN, 16), lambda i: (i, 0)),
        pl.BlockSpec((_BN, 1), lambda i: (i, 0)),
        pl.BlockSpec((_BN, _C), lambda i: (i, 0)),
        pl.BlockSpec((1, _C), lambda i: (0, 0)),
    ],
    out_specs=pl.BlockSpec((_BN, _C), lambda i: (i, 0)),
    out_shape=jax.ShapeDtypeStruct((_N, _C), jnp.float32),
)


def kernel(x, edge_index, W1, b1, W2, b2, Wg1, bg1, Wg2, bg2):
  src = edge_index[0]
  dst2 = edge_index[1].reshape(_E // _CH, _CH)
  ones = jnp.ones((_CH, 16), jnp.float32)
  b1r = b1.reshape(1, _H)
  b2r = b2.reshape(1, _C)
  bg1r = bg1.reshape(1, _H)
  bg2r = bg2.reshape(1, _C)

  degp = _deg_scatter(src, dst2, ones)
  xw, mlp = _tc_mm(x, W1, b1r, W2, b2r, Wg1)
  y, dinv = _tc_scale(degp, xw)
  hp = _pass1_scatter(src, dst2, y)
  up = _tc_mid(hp, y, dinv, bg1r, Wg2)
  sp = _pass2_scatter(src, dst2, up)
  return _tc_final(sp, up, dinv, mlp, bg2r)
